# Initial kernel scaffold; baseline (speedup 1.0000x reference)
import jax
import jax.numpy as jnp
from jax.experimental import pallas as pl


def kernel(x, edge_attr, edge_index, graph_attr, params):
    raise NotImplementedError("write your pallas kernel here")


# speedup vs baseline: 1.9620x; 1.9620x over previous
import functools

import jax
import jax.numpy as jnp
from jax import lax
from jax.experimental import pallas as pl
from jax.experimental.pallas import tpu as pltpu
from jax.experimental.pallas import tpu_sc as plsc

F32 = jnp.float32
_EPS_LN = 1e-5
_EPS_DIV = 1e-6


def _ln(v, g, b):
    mu = jnp.mean(v, axis=-1, keepdims=True)
    var = jnp.mean((v - mu) ** 2, axis=-1, keepdims=True)
    return (v - mu) * lax.rsqrt(var + _EPS_LN) * g + b


def _dot(a, b):
    return jnp.dot(a, b, preferred_element_type=F32)


def _mlp2(x, w1t, b1, w2t, b2, bm):
    m, k = x.shape
    f1 = w1t.shape[1]
    f2 = w2t.shape[1]

    def body(x_ref, w1_ref, b1_ref, w2_ref, b2_ref, o_ref):
        t = jax.nn.relu(_dot(x_ref[...], w1_ref[...]) + b1_ref[...])
        o_ref[...] = _dot(t, w2_ref[...]) + b2_ref[...]

    return pl.pallas_call(
        body,
        grid=(m // bm,),
        in_specs=[
            pl.BlockSpec((bm, k), lambda i: (i, 0)),
            pl.BlockSpec((k, f1), lambda i: (0, 0)),
            pl.BlockSpec((1, f1), lambda i: (0, 0)),
            pl.BlockSpec((f1, f2), lambda i: (0, 0)),
            pl.BlockSpec((1, f2), lambda i: (0, 0)),
        ],
        out_specs=pl.BlockSpec((bm, f2), lambda i: (i, 0)),
        out_shape=jax.ShapeDtypeStruct((m, f2), F32),
    )(x, w1t, b1.reshape(1, f1), w2t, b2.reshape(1, f2))


def _node_cat(h, wcat_t, bcat, bm):
    m, k = h.shape

    def body(h_ref, w_ref, b_ref, a1_ref, ts_ref, td_ref):
        acc = _dot(h_ref[...], w_ref[...]) + b_ref[...]
        a1_ref[...] = acc[:, 0:256]
        ts_ref[...] = acc[:, 256:1024]
        td_ref[...] = acc[:, 1024:1792]

    return pl.pallas_call(
        body,
        grid=(m // bm,),
        in_specs=[
            pl.BlockSpec((bm, k), lambda i: (i, 0)),
            pl.BlockSpec((k, 1792), lambda i: (0, 0)),
            pl.BlockSpec((1, 1792), lambda i: (0, 0)),
        ],
        out_specs=[
            pl.BlockSpec((bm, 256), lambda i: (i, 0)),
            pl.BlockSpec((bm, 768), lambda i: (i, 0)),
            pl.BlockSpec((bm, 768), lambda i: (i, 0)),
        ],
        out_shape=[
            jax.ShapeDtypeStruct((m, 256), F32),
            jax.ShapeDtypeStruct((m, 768), F32),
            jax.ShapeDtypeStruct((m, 768), F32),
        ],
    )(h, wcat_t, bcat.reshape(1, 1792))


def _edge_layer(e, gs, gd, b1t, b1b, lng, lnb, bm):
    m = e.shape[0]

    def body(e_ref, gs_ref, gd_ref, w_ref, b_ref, g_ref, gb_ref,
             efw_ref, mfw_ref, sfw_ref, mbw_ref, sbw_ref):
        ev = e_ref[...]
        gsv = gs_ref[...]
        gdv = gd_ref[...]
        b1h = _dot(ev, w_ref[...]) + b_ref[...]
        g = g_ref[...]
        gb = gb_ref[...]
        fw = ev + jax.nn.relu(_ln(b1h + gsv[:, 256:512] + gdv[:, 512:768], g, gb))
        bw = ev + jax.nn.relu(_ln(b1h + gdv[:, 256:512] + gsv[:, 512:768], g, gb))
        sfw = jax.nn.sigmoid(fw)
        sbw = jax.nn.sigmoid(bw)
        efw_ref[...] = fw
        mfw_ref[...] = gsv[:, 0:256] * sfw
        sfw_ref[...] = sfw
        mbw_ref[...] = gdv[:, 0:256] * sbw
        sbw_ref[...] = sbw

    outs = pl.pallas_call(
        body,
        grid=(m // bm,),
        in_specs=[
            pl.BlockSpec((bm, 256), lambda i: (i, 0)),
            pl.BlockSpec((bm, 768), lambda i: (i, 0)),
            pl.BlockSpec((bm, 768), lambda i: (i, 0)),
            pl.BlockSpec((256, 256), lambda i: (0, 0)),
            pl.BlockSpec((1, 256), lambda i: (0, 0)),
            pl.BlockSpec((1, 256), lambda i: (0, 0)),
            pl.BlockSpec((1, 256), lambda i: (0, 0)),
        ],
        out_specs=[pl.BlockSpec((bm, 256), lambda i: (i, 0))] * 5,
        out_shape=[jax.ShapeDtypeStruct((m, 256), F32)] * 5,
    )(e, gs, gd, b1t, b1b.reshape(1, 256), lng.reshape(1, 256),
      lnb.reshape(1, 256))
    return outs


def _node_update(a1h, smf, ssf, smb, ssb, h, lng, lnb, bm):
    m = h.shape[0]

    def body(a1_ref, smf_ref, ssf_ref, smb_ref, ssb_ref, h_ref, g_ref, b_ref,
             o_ref):
        hfw = smf_ref[...] / (ssf_ref[...] + _EPS_DIV)
        hbw = smb_ref[...] / (ssb_ref[...] + _EPS_DIV)
        hn = jax.nn.relu(_ln(a1_ref[...] + hfw + hbw, g_ref[...], b_ref[...]))
        o_ref[...] = h_ref[...] + hn

    return pl.pallas_call(
        body,
        grid=(m // bm,),
        in_specs=[pl.BlockSpec((bm, 256), lambda i: (i, 0))] * 6
        + [pl.BlockSpec((1, 256), lambda i: (0, 0))] * 2,
        out_specs=pl.BlockSpec((bm, 256), lambda i: (i, 0)),
        out_shape=jax.ShapeDtypeStruct((m, 256), F32),
    )(a1h, smf, ssf, smb, ssb, h, lng.reshape(1, 256), lnb.reshape(1, 256))


def _scorer(hs, hd, e, w1a, w1b, w1c, b1, s2pad, b2, bm):
    m = hs.shape[0]

    def body(hs_ref, hd_ref, e_ref, wa_ref, wb_ref, wc_ref, b1_ref, w2_ref,
             b2_ref, o_ref):
        t = (_dot(hs_ref[...], wa_ref[...]) + _dot(hd_ref[...], wb_ref[...])
             + _dot(e_ref[...], wc_ref[...]) + b1_ref[...])
        t = jax.nn.relu(t)
        s = _dot(t, w2_ref[...]) + b2_ref[...]
        o_ref[...] = jnp.maximum(s, 0.0)

    return pl.pallas_call(
        body,
        grid=(m // bm,),
        in_specs=[pl.BlockSpec((bm, 256), lambda i: (i, 0))] * 3
        + [pl.BlockSpec((256, 256), lambda i: (0, 0))] * 3
        + [pl.BlockSpec((1, 256), lambda i: (0, 0)),
           pl.BlockSpec((256, 128), lambda i: (0, 0)),
           pl.BlockSpec((1, 128), lambda i: (0, 0))],
        out_specs=pl.BlockSpec((bm, 128), lambda i: (i, 0)),
        out_shape=jax.ShapeDtypeStruct((m, 128), F32),
    )(hs, hd, e, w1a, w1b, w1c, b1.reshape(1, 256), s2pad, b2.reshape(1, 128))


_NC = 2
_NS = 16


def _sc_gather_pair(t1, t2, i1, i2):
    nrows, d = t1.shape
    e_ = i1.shape[0]
    nw = _NC * _NS
    per_w = e_ // nw
    ch = 40
    steps = per_w // ch
    mesh = plsc.VectorSubcoreMesh(core_axis_name="c", subcore_axis_name="s")

    @functools.partial(
        pl.kernel,
        mesh=mesh,
        out_type=(
            jax.ShapeDtypeStruct((e_, d), F32),
            jax.ShapeDtypeStruct((e_, d), F32),
        ),
        scratch_types=[
            pltpu.VMEM((ch,), jnp.int32),
            pltpu.VMEM((ch, d), F32),
            pltpu.SemaphoreType.DMA,
        ],
    )
    def k(t1_hbm, t2_hbm, i1_hbm, i2_hbm, o1_hbm, o2_hbm, idx_v, rows_v, sem):
        wid = lax.axis_index("s") * _NC + lax.axis_index("c")
        base = wid * per_w

        def body(i, carry):
            off = base + i * ch
            pltpu.sync_copy(i1_hbm.at[pl.ds(off, ch)], idx_v)
            pltpu.async_copy(t1_hbm.at[idx_v], rows_v, sem).wait()
            pltpu.sync_copy(rows_v, o1_hbm.at[pl.ds(off, ch)])
            pltpu.sync_copy(i2_hbm.at[pl.ds(off, ch)], idx_v)
            pltpu.async_copy(t2_hbm.at[idx_v], rows_v, sem).wait()
            pltpu.sync_copy(rows_v, o2_hbm.at[pl.ds(off, ch)])
            return carry

        lax.fori_loop(0, steps, body, 0)

    return k(t1, t2, i1, i2)


def _segsum(vals, idx, n_out, zeros_half):
    e_, d = vals.shape
    half = d // 2
    per_t = e_ // _NS
    ch = 80
    steps = per_t // ch
    row_step = 624
    rows_t = 640
    assert (_NS - 1) * row_step + rows_t == n_out
    mesh = plsc.VectorSubcoreMesh(core_axis_name="c", subcore_axis_name="s")

    @functools.partial(
        pl.kernel,
        mesh=mesh,
        out_type=jax.ShapeDtypeStruct((n_out, d), F32),
        scratch_types=[
            pltpu.VMEM((ch,), jnp.int32),
            pltpu.VMEM((ch, half), F32),
            pltpu.VMEM_SHARED((n_out, half), F32),
        ],
    )
    def k(vals_hbm, idx_hbm, z_hbm, out_hbm, idx_v, v_v, acc):
        c = lax.axis_index("c")
        s = lax.axis_index("s")
        r0 = s * row_step
        col0 = c * half
        pltpu.sync_copy(z_hbm.at[pl.ds(r0, rows_t)], acc.at[pl.ds(r0, rows_t)])
        plsc.subcore_barrier()
        base = s * per_t

        def body(i, carry):
            off = base + i * ch
            pltpu.sync_copy(idx_hbm.at[pl.ds(off, ch)], idx_v)
            pltpu.sync_copy(vals_hbm.at[pl.ds(off, ch), pl.ds(col0, half)], v_v)
            pltpu.sync_copy(v_v, acc.at[idx_v], add=True)
            return carry

        lax.fori_loop(0, steps, body, 0)
        plsc.subcore_barrier()
        pltpu.sync_copy(acc.at[pl.ds(r0, rows_t)],
                        out_hbm.at[pl.ds(r0, rows_t), pl.ds(col0, half)])

    return k(vals, idx, zeros_half)


def kernel(x, edge_attr, edge_index, graph_attr, params):
    del graph_attr
    p = params
    n = x.shape[0]
    src = edge_index[0]
    dst = edge_index[1]

    bm_n = 400
    bm_e = 320

    h = _mlp2(x, p["W11_w"].T, p["W11_b"], p["W12_w"].T, p["W12_b"], bm_n)
    e = _mlp2(edge_attr, p["W21_w"].T, p["W21_b"], p["W22_w"].T, p["W22_b"],
              bm_e)

    zeros_half = jnp.zeros((n, 128), F32)
    for l in range(3):
        pre = "L%d_" % l
        wcat_t = jnp.concatenate(
            [p[pre + nm + "_w"] for nm in
             ["A1", "A2", "B2", "B3", "A3", "B2", "B3"]], axis=0).T
        bcat = jnp.concatenate(
            [p[pre + nm + "_b"] for nm in
             ["A1", "A2", "B2", "B3", "A3", "B2", "B3"]], axis=0)
        a1h, t_src, t_dst = _node_cat(h, wcat_t, bcat, bm_n)
        gs, gd = _sc_gather_pair(t_src, t_dst, src, dst)
        e_fw, m_fw, s_fw, m_bw, s_bw = _edge_layer(
            e, gs, gd, p[pre + "B1_w"].T, p[pre + "B1_b"],
            p[pre + "lne_g"], p[pre + "lne_b"], bm_e)
        smf = _segsum(m_fw, dst, n, zeros_half)
        ssf = _segsum(s_fw, dst, n, zeros_half)
        smb = _segsum(m_bw, src, n, zeros_half)
        ssb = _segsum(s_bw, src, n, zeros_half)
        h = _node_update(a1h, smf, ssf, smb, ssb, h,
                         p[pre + "lnh_g"], p[pre + "lnh_b"], bm_n)
        e = e_fw

    hs, hd = _sc_gather_pair(h, h, src, dst)
    s1t = p["s1_w"].T
    s2pad = jnp.zeros((256, 128), F32).at[:, 0].set(p["s2_w"][0])
    b2pad = jnp.zeros((128,), F32) + p["s2_b"][0]
    score = _scorer(hs, hd, e, s1t[0:256], s1t[256:512], s1t[512:768],
                    p["s1_b"], s2pad, b2pad, bm_e)
    return score[:, 0:1]


# speedup vs baseline: 2.7262x; 1.3895x over previous
import functools

import jax
import jax.numpy as jnp
from jax import lax
from jax.experimental import pallas as pl
from jax.experimental.pallas import tpu as pltpu
from jax.experimental.pallas import tpu_sc as plsc

F32 = jnp.float32
_EPS_LN = 1e-5
_EPS_DIV = 1e-6


def _ln(v, g, b):
    mu = jnp.mean(v, axis=-1, keepdims=True)
    var = jnp.mean((v - mu) ** 2, axis=-1, keepdims=True)
    return (v - mu) * lax.rsqrt(var + _EPS_LN) * g + b


def _dot(a, b):
    return jnp.dot(a, b, preferred_element_type=F32)


def _mlp2(x, w1t, b1, w2t, b2, bm):
    m, k = x.shape
    f1 = w1t.shape[1]
    f2 = w2t.shape[1]

    def body(x_ref, w1_ref, b1_ref, w2_ref, b2_ref, o_ref):
        t = jax.nn.relu(_dot(x_ref[...], w1_ref[...]) + b1_ref[...])
        o_ref[...] = _dot(t, w2_ref[...]) + b2_ref[...]

    return pl.pallas_call(
        body,
        grid=(m // bm,),
        in_specs=[
            pl.BlockSpec((bm, k), lambda i: (i, 0)),
            pl.BlockSpec((k, f1), lambda i: (0, 0)),
            pl.BlockSpec((1, f1), lambda i: (0, 0)),
            pl.BlockSpec((f1, f2), lambda i: (0, 0)),
            pl.BlockSpec((1, f2), lambda i: (0, 0)),
        ],
        out_specs=pl.BlockSpec((bm, f2), lambda i: (i, 0)),
        out_shape=jax.ShapeDtypeStruct((m, f2), F32),
    )(x, w1t, b1.reshape(1, f1), w2t, b2.reshape(1, f2))


def _node_cat(h, wcat_t, bcat, bm):
    m, k = h.shape

    def body(h_ref, w_ref, b_ref, a1_ref, ts_ref, td_ref):
        acc = _dot(h_ref[...], w_ref[...]) + b_ref[...]
        a1_ref[...] = acc[:, 0:256]
        ts_ref[...] = acc[:, 256:1024]
        td_ref[...] = acc[:, 1024:1792]

    return pl.pallas_call(
        body,
        grid=(m // bm,),
        in_specs=[
            pl.BlockSpec((bm, k), lambda i: (i, 0)),
            pl.BlockSpec((k, 1792), lambda i: (0, 0)),
            pl.BlockSpec((1, 1792), lambda i: (0, 0)),
        ],
        out_specs=[
            pl.BlockSpec((bm, 256), lambda i: (i, 0)),
            pl.BlockSpec((bm, 768), lambda i: (i, 0)),
            pl.BlockSpec((bm, 768), lambda i: (i, 0)),
        ],
        out_shape=[
            jax.ShapeDtypeStruct((m, 256), F32),
            jax.ShapeDtypeStruct((m, 768), F32),
            jax.ShapeDtypeStruct((m, 768), F32),
        ],
    )(h, wcat_t, bcat.reshape(1, 1792))


def _edge_layer(e, gs, gd, b1t, b1b, lng, lnb, bm):
    m = e.shape[0]

    def body(e_ref, gs_ref, gd_ref, w_ref, b_ref, g_ref, gb_ref,
             efw_ref, mfw_ref, sfw_ref, mbw_ref, sbw_ref):
        ev = e_ref[...]
        gsv = gs_ref[...]
        gdv = gd_ref[...]
        b1h = _dot(ev, w_ref[...]) + b_ref[...]
        g = g_ref[...]
        gb = gb_ref[...]
        fw = ev + jax.nn.relu(_ln(b1h + gsv[:, 256:512] + gdv[:, 512:768], g, gb))
        bw = ev + jax.nn.relu(_ln(b1h + gdv[:, 256:512] + gsv[:, 512:768], g, gb))
        sfw = jax.nn.sigmoid(fw)
        sbw = jax.nn.sigmoid(bw)
        efw_ref[...] = fw
        mfw_ref[...] = gsv[:, 0:256] * sfw
        sfw_ref[...] = sfw
        mbw_ref[...] = gdv[:, 0:256] * sbw
        sbw_ref[...] = sbw

    outs = pl.pallas_call(
        body,
        grid=(m // bm,),
        in_specs=[
            pl.BlockSpec((bm, 256), lambda i: (i, 0)),
            pl.BlockSpec((bm, 768), lambda i: (i, 0)),
            pl.BlockSpec((bm, 768), lambda i: (i, 0)),
            pl.BlockSpec((256, 256), lambda i: (0, 0)),
            pl.BlockSpec((1, 256), lambda i: (0, 0)),
            pl.BlockSpec((1, 256), lambda i: (0, 0)),
            pl.BlockSpec((1, 256), lambda i: (0, 0)),
        ],
        out_specs=[pl.BlockSpec((bm, 256), lambda i: (i, 0))] * 5,
        out_shape=[jax.ShapeDtypeStruct((m, 256), F32)] * 5,
    )(e, gs, gd, b1t, b1b.reshape(1, 256), lng.reshape(1, 256),
      lnb.reshape(1, 256))
    return outs


def _node_update(a1h, smf, ssf, smb, ssb, h, lng, lnb, bm):
    m = h.shape[0]

    def body(a1_ref, smf_ref, ssf_ref, smb_ref, ssb_ref, h_ref, g_ref, b_ref,
             o_ref):
        hfw = smf_ref[...] / (ssf_ref[...] + _EPS_DIV)
        hbw = smb_ref[...] / (ssb_ref[...] + _EPS_DIV)
        hn = jax.nn.relu(_ln(a1_ref[...] + hfw + hbw, g_ref[...], b_ref[...]))
        o_ref[...] = h_ref[...] + hn

    return pl.pallas_call(
        body,
        grid=(m // bm,),
        in_specs=[pl.BlockSpec((bm, 256), lambda i: (i, 0))] * 6
        + [pl.BlockSpec((1, 256), lambda i: (0, 0))] * 2,
        out_specs=pl.BlockSpec((bm, 256), lambda i: (i, 0)),
        out_shape=jax.ShapeDtypeStruct((m, 256), F32),
    )(a1h, smf, ssf, smb, ssb, h, lng.reshape(1, 256), lnb.reshape(1, 256))


def _scorer(hs, hd, e, w1a, w1b, w1c, b1, s2pad, b2, bm):
    m = hs.shape[0]

    def body(hs_ref, hd_ref, e_ref, wa_ref, wb_ref, wc_ref, b1_ref, w2_ref,
             b2_ref, o_ref):
        t = (_dot(hs_ref[...], wa_ref[...]) + _dot(hd_ref[...], wb_ref[...])
             + _dot(e_ref[...], wc_ref[...]) + b1_ref[...])
        t = jax.nn.relu(t)
        s = _dot(t, w2_ref[...]) + b2_ref[...]
        o_ref[...] = jnp.maximum(s, 0.0)

    return pl.pallas_call(
        body,
        grid=(m // bm,),
        in_specs=[pl.BlockSpec((bm, 256), lambda i: (i, 0))] * 3
        + [pl.BlockSpec((256, 256), lambda i: (0, 0))] * 3
        + [pl.BlockSpec((1, 256), lambda i: (0, 0)),
           pl.BlockSpec((256, 128), lambda i: (0, 0)),
           pl.BlockSpec((1, 128), lambda i: (0, 0))],
        out_specs=pl.BlockSpec((bm, 128), lambda i: (i, 0)),
        out_shape=jax.ShapeDtypeStruct((m, 128), F32),
    )(hs, hd, e, w1a, w1b, w1c, b1.reshape(1, 256), s2pad, b2.reshape(1, 128))


_NC = 2
_NS = 16


_GCH = 40


def _sc_gather_pair(t1, t2, i1, i2):
    nrows, d = t1.shape
    e_ = i1.shape[0]
    nw = _NC * _NS
    per_w = e_ // nw
    ch = _GCH
    steps = per_w // ch
    assert steps % 2 == 1
    i1r = i1.reshape(nw, steps, ch)
    i2r = i2.reshape(nw, steps, ch)
    mesh = plsc.VectorSubcoreMesh(core_axis_name="c", subcore_axis_name="s")

    @functools.partial(
        pl.kernel,
        mesh=mesh,
        out_type=(
            jax.ShapeDtypeStruct((e_, d), F32),
            jax.ShapeDtypeStruct((e_, d), F32),
        ),
        scratch_types=[
            pltpu.VMEM((steps, ch), jnp.int32),
            pltpu.VMEM((ch, d), F32),
            pltpu.VMEM((ch, d), F32),
            pltpu.SemaphoreType.DMA,
            pltpu.SemaphoreType.DMA,
        ],
    )
    def k(t1_hbm, t2_hbm, i1_hbm, i2_hbm, o1_hbm, o2_hbm, idx_v, ra, rb,
          sema, semb):
        wid = lax.axis_index("s") * _NC + lax.axis_index("c")
        base = wid * per_w

        def gather_one(t_hbm, i_hbm, o_hbm):
            pltpu.sync_copy(i_hbm.at[wid], idx_v)
            pltpu.async_copy(t_hbm.at[idx_v.at[0]], ra, sema)

            def body(j, carry):
                c0 = 2 * j
                pltpu.async_copy(t_hbm.at[idx_v.at[c0 + 1]], rb, semb)
                pltpu.make_async_copy(t_hbm.at[idx_v.at[c0]], ra, sema).wait()
                pltpu.sync_copy(ra, o_hbm.at[pl.ds(base + c0 * ch, ch)])
                pltpu.async_copy(t_hbm.at[idx_v.at[c0 + 2]], ra, sema)
                pltpu.make_async_copy(t_hbm.at[idx_v.at[c0 + 1]], rb,
                                      semb).wait()
                pltpu.sync_copy(rb, o_hbm.at[pl.ds(base + (c0 + 1) * ch, ch)])
                return carry

            lax.fori_loop(0, (steps - 1) // 2, body, 0)
            last = steps - 1
            pltpu.make_async_copy(t_hbm.at[idx_v.at[last]], ra, sema).wait()
            pltpu.sync_copy(ra, o_hbm.at[pl.ds(base + last * ch, ch)])

        gather_one(t1_hbm, i1_hbm, o1_hbm)
        gather_one(t2_hbm, i2_hbm, o2_hbm)

    return k(t1, t2, i1r, i2r)


def _segsum(vals, idx, n_out, zeros_half):
    e_, d = vals.shape
    half = d // 2
    per_t = e_ // _NS
    ch = 80
    steps = per_t // ch
    assert steps % 2 == 1
    idxr = idx.reshape(_NS, steps, ch)
    row_step = 624
    rows_t = 640
    assert (_NS - 1) * row_step + rows_t == n_out
    mesh = plsc.VectorSubcoreMesh(core_axis_name="c", subcore_axis_name="s")

    @functools.partial(
        pl.kernel,
        mesh=mesh,
        out_type=jax.ShapeDtypeStruct((n_out, d), F32),
        scratch_types=[
            pltpu.VMEM((steps, ch), jnp.int32),
            pltpu.VMEM((ch, half), F32),
            pltpu.VMEM((ch, half), F32),
            pltpu.VMEM_SHARED((n_out, half), F32),
            pltpu.SemaphoreType.DMA,
            pltpu.SemaphoreType.DMA,
        ],
    )
    def k(vals_hbm, idx_hbm, z_hbm, out_hbm, idx_v, va, vb, acc, sema, semb):
        c = lax.axis_index("c")
        s = lax.axis_index("s")
        r0 = s * row_step
        col0 = c * half
        pltpu.sync_copy(z_hbm.at[pl.ds(r0, rows_t)], acc.at[pl.ds(r0, rows_t)])
        pltpu.sync_copy(idx_hbm.at[s], idx_v)
        plsc.subcore_barrier()
        base = s * per_t

        def vslice(cc):
            return vals_hbm.at[pl.ds(base + cc * ch, ch), pl.ds(col0, half)]

        pltpu.async_copy(vslice(0), va, sema)

        def body(j, carry):
            c0 = 2 * j
            pltpu.async_copy(vslice(c0 + 1), vb, semb)
            pltpu.make_async_copy(vslice(c0), va, sema).wait()
            pltpu.sync_copy(va, acc.at[idx_v.at[c0]], add=True)
            pltpu.async_copy(vslice(c0 + 2), va, sema)
            pltpu.make_async_copy(vslice(c0 + 1), vb, semb).wait()
            pltpu.sync_copy(vb, acc.at[idx_v.at[c0 + 1]], add=True)
            return carry

        lax.fori_loop(0, (steps - 1) // 2, body, 0)
        last = steps - 1
        pltpu.make_async_copy(vslice(last), va, sema).wait()
        pltpu.sync_copy(va, acc.at[idx_v.at[last]], add=True)
        plsc.subcore_barrier()
        pltpu.sync_copy(acc.at[pl.ds(r0, rows_t)],
                        out_hbm.at[pl.ds(r0, rows_t), pl.ds(col0, half)])

    return k(vals, idxr, zeros_half)


def kernel(x, edge_attr, edge_index, graph_attr, params):
    del graph_attr
    p = params
    n = x.shape[0]
    src = edge_index[0]
    dst = edge_index[1]

    bm_n = 400
    bm_e = 320

    h = _mlp2(x, p["W11_w"].T, p["W11_b"], p["W12_w"].T, p["W12_b"], bm_n)
    e = _mlp2(edge_attr, p["W21_w"].T, p["W21_b"], p["W22_w"].T, p["W22_b"],
              bm_e)

    zeros_half = jnp.zeros((n, 128), F32)
    for l in range(3):
        pre = "L%d_" % l
        wcat_t = jnp.concatenate(
            [p[pre + nm + "_w"] for nm in
             ["A1", "A2", "B2", "B3", "A3", "B2", "B3"]], axis=0).T
        bcat = jnp.concatenate(
            [p[pre + nm + "_b"] for nm in
             ["A1", "A2", "B2", "B3", "A3", "B2", "B3"]], axis=0)
        a1h, t_src, t_dst = _node_cat(h, wcat_t, bcat, bm_n)
        gs, gd = _sc_gather_pair(t_src, t_dst, src, dst)
        e_fw, m_fw, s_fw, m_bw, s_bw = _edge_layer(
            e, gs, gd, p[pre + "B1_w"].T, p[pre + "B1_b"],
            p[pre + "lne_g"], p[pre + "lne_b"], bm_e)
        smf = _segsum(m_fw, dst, n, zeros_half)
        ssf = _segsum(s_fw, dst, n, zeros_half)
        smb = _segsum(m_bw, src, n, zeros_half)
        ssb = _segsum(s_bw, src, n, zeros_half)
        h = _node_update(a1h, smf, ssf, smb, ssb, h,
                         p[pre + "lnh_g"], p[pre + "lnh_b"], bm_n)
        e = e_fw

    hs, hd = _sc_gather_pair(h, h, src, dst)
    s1t = p["s1_w"].T
    s2pad = jnp.zeros((256, 128), F32).at[:, 0].set(p["s2_w"][0])
    b2pad = jnp.zeros((128,), F32) + p["s2_b"][0]
    score = _scorer(hs, hd, e, s1t[0:256], s1t[256:512], s1t[512:768],
                    p["s1_b"], s2pad, b2pad, bm_e)
    return score[:, 0:1]


# speedup vs baseline: 2.8001x; 1.0271x over previous
import functools

import jax
import jax.numpy as jnp
from jax import lax
from jax.experimental import pallas as pl
from jax.experimental.pallas import tpu as pltpu
from jax.experimental.pallas import tpu_sc as plsc

F32 = jnp.float32
_EPS_LN = 1e-5
_EPS_DIV = 1e-6


def _ln(v, g, b):
    mu = jnp.mean(v, axis=-1, keepdims=True)
    var = jnp.mean((v - mu) ** 2, axis=-1, keepdims=True)
    return (v - mu) * lax.rsqrt(var + _EPS_LN) * g + b


def _dot(a, b):
    return jnp.dot(a, b, preferred_element_type=F32)


def _mlp2(x, w1t, b1, w2t, b2, bm):
    m, k = x.shape
    f1 = w1t.shape[1]
    f2 = w2t.shape[1]

    def body(x_ref, w1_ref, b1_ref, w2_ref, b2_ref, o_ref):
        t = jax.nn.relu(_dot(x_ref[...], w1_ref[...]) + b1_ref[...])
        o_ref[...] = _dot(t, w2_ref[...]) + b2_ref[...]

    return pl.pallas_call(
        body,
        grid=(m // bm,),
        in_specs=[
            pl.BlockSpec((bm, k), lambda i: (i, 0)),
            pl.BlockSpec((k, f1), lambda i: (0, 0)),
            pl.BlockSpec((1, f1), lambda i: (0, 0)),
            pl.BlockSpec((f1, f2), lambda i: (0, 0)),
            pl.BlockSpec((1, f2), lambda i: (0, 0)),
        ],
        out_specs=pl.BlockSpec((bm, f2), lambda i: (i, 0)),
        out_shape=jax.ShapeDtypeStruct((m, f2), F32),
    )(x, w1t, b1.reshape(1, f1), w2t, b2.reshape(1, f2))


def _node_cat(h, wcat_t, bcat, bm):
    m, k = h.shape

    def body(h_ref, w_ref, b_ref, a1_ref, ts_ref, td_ref):
        acc = _dot(h_ref[...], w_ref[...]) + b_ref[...]
        a1_ref[...] = acc[:, 0:256]
        ts_ref[...] = acc[:, 256:1024]
        td_ref[...] = acc[:, 1024:1792]

    return pl.pallas_call(
        body,
        grid=(m // bm,),
        in_specs=[
            pl.BlockSpec((bm, k), lambda i: (i, 0)),
            pl.BlockSpec((k, 1792), lambda i: (0, 0)),
            pl.BlockSpec((1, 1792), lambda i: (0, 0)),
        ],
        out_specs=[
            pl.BlockSpec((bm, 256), lambda i: (i, 0)),
            pl.BlockSpec((bm, 768), lambda i: (i, 0)),
            pl.BlockSpec((bm, 768), lambda i: (i, 0)),
        ],
        out_shape=[
            jax.ShapeDtypeStruct((m, 256), F32),
            jax.ShapeDtypeStruct((m, 768), F32),
            jax.ShapeDtypeStruct((m, 768), F32),
        ],
    )(h, wcat_t, bcat.reshape(1, 1792))


def _edge_layer(e, gs, gd, b1t, b1b, lng, lnb, bm):
    m = e.shape[0]

    def body(e_ref, gs_ref, gd_ref, w_ref, b_ref, g_ref, gb_ref,
             efw_ref, mfw_ref, sfw_ref, mbw_ref, sbw_ref):
        ev = e_ref[...]
        gsv = gs_ref[...]
        gdv = gd_ref[...]
        b1h = _dot(ev, w_ref[...]) + b_ref[...]
        g = g_ref[...]
        gb = gb_ref[...]
        fw = ev + jax.nn.relu(_ln(b1h + gsv[:, 256:512] + gdv[:, 512:768], g, gb))
        bw = ev + jax.nn.relu(_ln(b1h + gdv[:, 256:512] + gsv[:, 512:768], g, gb))
        sfw = jax.nn.sigmoid(fw)
        sbw = jax.nn.sigmoid(bw)
        efw_ref[...] = fw
        mfw_ref[...] = gsv[:, 0:256] * sfw
        sfw_ref[...] = sfw
        mbw_ref[...] = gdv[:, 0:256] * sbw
        sbw_ref[...] = sbw

    outs = pl.pallas_call(
        body,
        grid=(m // bm,),
        in_specs=[
            pl.BlockSpec((bm, 256), lambda i: (i, 0)),
            pl.BlockSpec((bm, 768), lambda i: (i, 0)),
            pl.BlockSpec((bm, 768), lambda i: (i, 0)),
            pl.BlockSpec((256, 256), lambda i: (0, 0)),
            pl.BlockSpec((1, 256), lambda i: (0, 0)),
            pl.BlockSpec((1, 256), lambda i: (0, 0)),
            pl.BlockSpec((1, 256), lambda i: (0, 0)),
        ],
        out_specs=[pl.BlockSpec((bm, 256), lambda i: (i, 0))] * 5,
        out_shape=[jax.ShapeDtypeStruct((m, 256), F32)] * 5,
    )(e, gs, gd, b1t, b1b.reshape(1, 256), lng.reshape(1, 256),
      lnb.reshape(1, 256))
    return outs


def _node_update(a1h, smf, ssf, smb, ssb, h, lng, lnb, bm):
    m = h.shape[0]

    def body(a1_ref, smf_ref, ssf_ref, smb_ref, ssb_ref, h_ref, g_ref, b_ref,
             o_ref):
        hfw = smf_ref[...] / (ssf_ref[...] + _EPS_DIV)
        hbw = smb_ref[...] / (ssb_ref[...] + _EPS_DIV)
        hn = jax.nn.relu(_ln(a1_ref[...] + hfw + hbw, g_ref[...], b_ref[...]))
        o_ref[...] = h_ref[...] + hn

    return pl.pallas_call(
        body,
        grid=(m // bm,),
        in_specs=[pl.BlockSpec((bm, 256), lambda i: (i, 0))] * 6
        + [pl.BlockSpec((1, 256), lambda i: (0, 0))] * 2,
        out_specs=pl.BlockSpec((bm, 256), lambda i: (i, 0)),
        out_shape=jax.ShapeDtypeStruct((m, 256), F32),
    )(a1h, smf, ssf, smb, ssb, h, lng.reshape(1, 256), lnb.reshape(1, 256))


def _scorer(hs, hd, e, w1a, w1b, w1c, b1, s2pad, b2, bm):
    m = hs.shape[0]

    def body(hs_ref, hd_ref, e_ref, wa_ref, wb_ref, wc_ref, b1_ref, w2_ref,
             b2_ref, o_ref):
        t = (_dot(hs_ref[...], wa_ref[...]) + _dot(hd_ref[...], wb_ref[...])
             + _dot(e_ref[...], wc_ref[...]) + b1_ref[...])
        t = jax.nn.relu(t)
        s = _dot(t, w2_ref[...]) + b2_ref[...]
        o_ref[...] = jnp.maximum(s, 0.0)

    return pl.pallas_call(
        body,
        grid=(m // bm,),
        in_specs=[pl.BlockSpec((bm, 256), lambda i: (i, 0))] * 3
        + [pl.BlockSpec((256, 256), lambda i: (0, 0))] * 3
        + [pl.BlockSpec((1, 256), lambda i: (0, 0)),
           pl.BlockSpec((256, 128), lambda i: (0, 0)),
           pl.BlockSpec((1, 128), lambda i: (0, 0))],
        out_specs=pl.BlockSpec((bm, 128), lambda i: (i, 0)),
        out_shape=jax.ShapeDtypeStruct((m, 128), F32),
    )(hs, hd, e, w1a, w1b, w1c, b1.reshape(1, 256), s2pad, b2.reshape(1, 128))


_NC = 2
_NS = 16


_GCH = 40


def _pipe(steps, issue, wait_consume):
    issue(0, 0)

    def body(j, carry):
        c0 = 2 * j
        issue(c0 + 1, 1)
        wait_consume(c0, 0)

        @pl.when(c0 + 2 < steps)
        def _():
            issue(c0 + 2, 0)

        wait_consume(c0 + 1, 1)
        return carry

    if steps // 2 > 0:
        lax.fori_loop(0, steps // 2, body, 0)
    if steps % 2:
        wait_consume(steps - 1, 0)


def _sc_gather_pair(t1, t2, i1, i2):
    nrows, d = t1.shape
    e_ = i1.shape[0]
    nw = _NC * _NS
    per_w = e_ // nw
    ch = _GCH
    steps = per_w // ch
    i1r = i1.reshape(nw, steps, ch)
    i2r = i2.reshape(nw, steps, ch)
    mesh = plsc.VectorSubcoreMesh(core_axis_name="c", subcore_axis_name="s")

    @functools.partial(
        pl.kernel,
        mesh=mesh,
        out_type=(
            jax.ShapeDtypeStruct((e_, d), F32),
            jax.ShapeDtypeStruct((e_, d), F32),
        ),
        scratch_types=[
            pltpu.VMEM((steps, ch), jnp.int32),
            pltpu.VMEM((ch, d), F32),
            pltpu.VMEM((ch, d), F32),
            pltpu.SemaphoreType.DMA,
            pltpu.SemaphoreType.DMA,
        ],
    )
    def k(t1_hbm, t2_hbm, i1_hbm, i2_hbm, o1_hbm, o2_hbm, idx_v, ra, rb,
          sema, semb):
        wid = lax.axis_index("s") * _NC + lax.axis_index("c")
        base = wid * per_w

        def gather_one(t_hbm, i_hbm, o_hbm):
            pltpu.sync_copy(i_hbm.at[wid], idx_v)
            bufs = (ra, rb)
            sems = (sema, semb)

            def issue(c, b):
                pltpu.async_copy(t_hbm.at[idx_v.at[c]], bufs[b], sems[b])

            def wait_consume(c, b):
                pltpu.make_async_copy(t_hbm.at[idx_v.at[c]], bufs[b],
                                      sems[b]).wait()
                pltpu.sync_copy(bufs[b], o_hbm.at[pl.ds(base + c * ch, ch)])

            _pipe(steps, issue, wait_consume)

        gather_one(t1_hbm, i1_hbm, o1_hbm)
        gather_one(t2_hbm, i2_hbm, o2_hbm)

    return k(t1, t2, i1r, i2r)


def _segsum2(vals_a, idx_a, vals_b, idx_b, n_out, zeros_half):
    d = vals_a.shape[1]
    half = d // 2
    ch = 80
    steps_a = vals_a.shape[0] // _NS // ch
    steps_b = vals_b.shape[0] // _NS // ch
    steps_mx = max(steps_a, steps_b)
    idx_ar = idx_a.reshape(_NS, steps_a, ch)
    idx_br = idx_b.reshape(_NS, steps_b, ch)
    row_step = 624
    rows_t = 640
    assert (_NS - 1) * row_step + rows_t == n_out
    mesh = plsc.VectorSubcoreMesh(core_axis_name="c", subcore_axis_name="s")

    @functools.partial(
        pl.kernel,
        mesh=mesh,
        out_type=jax.ShapeDtypeStruct((n_out, d), F32),
        scratch_types=[
            pltpu.VMEM((steps_mx, ch), jnp.int32),
            pltpu.VMEM((ch, half), F32),
            pltpu.VMEM((ch, half), F32),
            pltpu.VMEM_SHARED((n_out, half), F32),
            pltpu.SemaphoreType.DMA,
            pltpu.SemaphoreType.DMA,
        ],
    )
    def k(va_hbm, ia_hbm, vb_hbm, ib_hbm, z_hbm, out_hbm, idx_v, va, vb, acc,
          sema, semb):
        c = lax.axis_index("c")
        s = lax.axis_index("s")
        r0 = s * row_step
        col0 = c * half
        pltpu.sync_copy(z_hbm.at[pl.ds(r0, rows_t)], acc.at[pl.ds(r0, rows_t)])
        plsc.subcore_barrier()
        bufs = (va, vb)
        sems = (sema, semb)

        def accumulate(vals_hbm, i_hbm, steps):
            pltpu.sync_copy(i_hbm.at[s], idx_v.at[pl.ds(0, steps)])
            per_t = steps * ch
            base = s * per_t

            def vslice(cc):
                return vals_hbm.at[pl.ds(base + cc * ch, ch),
                                   pl.ds(col0, half)]

            def issue(cc, b):
                pltpu.async_copy(vslice(cc), bufs[b], sems[b])

            def wait_consume(cc, b):
                pltpu.make_async_copy(vslice(cc), bufs[b], sems[b]).wait()
                pltpu.sync_copy(bufs[b], acc.at[idx_v.at[cc]], add=True)

            _pipe(steps, issue, wait_consume)

        accumulate(va_hbm, ia_hbm, steps_a)
        accumulate(vb_hbm, ib_hbm, steps_b)
        plsc.subcore_barrier()
        pltpu.sync_copy(acc.at[pl.ds(r0, rows_t)],
                        out_hbm.at[pl.ds(r0, rows_t), pl.ds(col0, half)])

    return k(vals_a, idx_ar, vals_b, idx_br, zeros_half)


def kernel(x, edge_attr, edge_index, graph_attr, params):
    del graph_attr
    p = params
    n = x.shape[0]
    e_total = edge_index.shape[1]
    ea = 81920
    eb = e_total - ea
    src_a, src_b = edge_index[0, :ea], edge_index[0, ea:]
    dst_a, dst_b = edge_index[1, :ea], edge_index[1, ea:]

    bm_n = 400
    bm_e = 320

    h = _mlp2(x, p["W11_w"].T, p["W11_b"], p["W12_w"].T, p["W12_b"], bm_n)
    e_a = _mlp2(edge_attr[:ea], p["W21_w"].T, p["W21_b"], p["W22_w"].T,
                p["W22_b"], bm_e)
    e_b = _mlp2(edge_attr[ea:], p["W21_w"].T, p["W21_b"], p["W22_w"].T,
                p["W22_b"], bm_e)

    zeros_half = jnp.zeros((n, 128), F32)
    for l in range(3):
        pre = "L%d_" % l
        wcat_t = jnp.concatenate(
            [p[pre + nm + "_w"] for nm in
             ["A1", "A2", "B2", "B3", "A3", "B2", "B3"]], axis=0).T
        bcat = jnp.concatenate(
            [p[pre + nm + "_b"] for nm in
             ["A1", "A2", "B2", "B3", "A3", "B2", "B3"]], axis=0)
        a1h, t_src, t_dst = _node_cat(h, wcat_t, bcat, bm_n)
        gs_a, gd_a = _sc_gather_pair(t_src, t_dst, src_a, dst_a)
        gs_b, gd_b = _sc_gather_pair(t_src, t_dst, src_b, dst_b)
        efw_a, mf_a, sf_a, mb_a, sb_a = _edge_layer(
            e_a, gs_a, gd_a, p[pre + "B1_w"].T, p[pre + "B1_b"],
            p[pre + "lne_g"], p[pre + "lne_b"], bm_e)
        efw_b, mf_b, sf_b, mb_b, sb_b = _edge_layer(
            e_b, gs_b, gd_b, p[pre + "B1_w"].T, p[pre + "B1_b"],
            p[pre + "lne_g"], p[pre + "lne_b"], bm_e)
        smf = _segsum2(mf_a, dst_a, mf_b, dst_b, n, zeros_half)
        ssf = _segsum2(sf_a, dst_a, sf_b, dst_b, n, zeros_half)
        smb = _segsum2(mb_a, src_a, mb_b, src_b, n, zeros_half)
        ssb = _segsum2(sb_a, src_a, sb_b, src_b, n, zeros_half)
        h = _node_update(a1h, smf, ssf, smb, ssb, h,
                         p[pre + "lnh_g"], p[pre + "lnh_b"], bm_n)
        e_a, e_b = efw_a, efw_b

    hs_a, hd_a = _sc_gather_pair(h, h, src_a, dst_a)
    hs_b, hd_b = _sc_gather_pair(h, h, src_b, dst_b)
    s1t = p["s1_w"].T
    s2pad = jnp.zeros((256, 128), F32).at[:, 0].set(p["s2_w"][0])
    b2pad = jnp.zeros((128,), F32) + p["s2_b"][0]
    score_a = _scorer(hs_a, hd_a, e_a, s1t[0:256], s1t[256:512], s1t[512:768],
                      p["s1_b"], s2pad, b2pad, bm_e)
    score_b = _scorer(hs_b, hd_b, e_b, s1t[0:256], s1t[256:512], s1t[512:768],
                      p["s1_b"], s2pad, b2pad, bm_e)
    return jnp.concatenate([score_a[:, 0:1], score_b[:, 0:1]], axis=0)


# speedup vs baseline: 2.8176x; 1.0063x over previous
import functools

import jax
import jax.numpy as jnp
from jax import lax
from jax.experimental import pallas as pl
from jax.experimental.pallas import tpu as pltpu
from jax.experimental.pallas import tpu_sc as plsc

F32 = jnp.float32
_EPS_LN = 1e-5
_EPS_DIV = 1e-6


def _ln(v, g, b):
    mu = jnp.mean(v, axis=-1, keepdims=True)
    var = jnp.mean((v - mu) ** 2, axis=-1, keepdims=True)
    return (v - mu) * lax.rsqrt(var + _EPS_LN) * g + b


def _dot(a, b):
    return jnp.dot(a, b, preferred_element_type=F32)


def _mlp2(x, w1t, b1, w2t, b2, bm):
    m, k = x.shape
    f1 = w1t.shape[1]
    f2 = w2t.shape[1]

    def body(x_ref, w1_ref, b1_ref, w2_ref, b2_ref, o_ref):
        t = jax.nn.relu(_dot(x_ref[...], w1_ref[...]) + b1_ref[...])
        o_ref[...] = _dot(t, w2_ref[...]) + b2_ref[...]

    return pl.pallas_call(
        body,
        grid=(m // bm,),
        in_specs=[
            pl.BlockSpec((bm, k), lambda i: (i, 0)),
            pl.BlockSpec((k, f1), lambda i: (0, 0)),
            pl.BlockSpec((1, f1), lambda i: (0, 0)),
            pl.BlockSpec((f1, f2), lambda i: (0, 0)),
            pl.BlockSpec((1, f2), lambda i: (0, 0)),
        ],
        out_specs=pl.BlockSpec((bm, f2), lambda i: (i, 0)),
        out_shape=jax.ShapeDtypeStruct((m, f2), F32),
    )(x, w1t, b1.reshape(1, f1), w2t, b2.reshape(1, f2))


def _node_cat(h, wcat_t, bcat, bm):
    m, k = h.shape

    def body(h_ref, w_ref, b_ref, a1_ref, ts_ref, td_ref):
        acc = _dot(h_ref[...], w_ref[...]) + b_ref[...]
        a1_ref[...] = acc[:, 0:256]
        ts_ref[...] = acc[:, 256:1024]
        td_ref[...] = acc[:, 1024:1792]

    return pl.pallas_call(
        body,
        grid=(m // bm,),
        in_specs=[
            pl.BlockSpec((bm, k), lambda i: (i, 0)),
            pl.BlockSpec((k, 1792), lambda i: (0, 0)),
            pl.BlockSpec((1, 1792), lambda i: (0, 0)),
        ],
        out_specs=[
            pl.BlockSpec((bm, 256), lambda i: (i, 0)),
            pl.BlockSpec((bm, 768), lambda i: (i, 0)),
            pl.BlockSpec((bm, 768), lambda i: (i, 0)),
        ],
        out_shape=[
            jax.ShapeDtypeStruct((m, 256), F32),
            jax.ShapeDtypeStruct((m, 768), F32),
            jax.ShapeDtypeStruct((m, 768), F32),
        ],
    )(h, wcat_t, bcat.reshape(1, 1792))


def _edge_layer(e, gs, gd, b1t, b1b, lng, lnb, bm):
    m = e.shape[0]

    def body(e_ref, gs_ref, gd_ref, w_ref, b_ref, g_ref, gb_ref,
             efw_ref, mfw_ref, sfw_ref, mbw_ref, sbw_ref):
        ev = e_ref[...]
        gsv = gs_ref[...]
        gdv = gd_ref[...]
        b1h = _dot(ev, w_ref[...]) + b_ref[...]
        g = g_ref[...]
        gb = gb_ref[...]
        fw = ev + jax.nn.relu(_ln(b1h + gsv[:, 256:512] + gdv[:, 512:768], g, gb))
        bw = ev + jax.nn.relu(_ln(b1h + gdv[:, 256:512] + gsv[:, 512:768], g, gb))
        sfw = jax.nn.sigmoid(fw)
        sbw = jax.nn.sigmoid(bw)
        efw_ref[...] = fw
        mfw_ref[...] = gsv[:, 0:256] * sfw
        sfw_ref[...] = sfw
        mbw_ref[...] = gdv[:, 0:256] * sbw
        sbw_ref[...] = sbw

    outs = pl.pallas_call(
        body,
        grid=(m // bm,),
        in_specs=[
            pl.BlockSpec((bm, 256), lambda i: (i, 0)),
            pl.BlockSpec((bm, 768), lambda i: (i, 0)),
            pl.BlockSpec((bm, 768), lambda i: (i, 0)),
            pl.BlockSpec((256, 256), lambda i: (0, 0)),
            pl.BlockSpec((1, 256), lambda i: (0, 0)),
            pl.BlockSpec((1, 256), lambda i: (0, 0)),
            pl.BlockSpec((1, 256), lambda i: (0, 0)),
        ],
        out_specs=[pl.BlockSpec((bm, 256), lambda i: (i, 0))] * 5,
        out_shape=[jax.ShapeDtypeStruct((m, 256), F32)] * 5,
    )(e, gs, gd, b1t, b1b.reshape(1, 256), lng.reshape(1, 256),
      lnb.reshape(1, 256))
    return outs


def _node_update(a1h, smf, ssf, smb, ssb, h, lng, lnb, bm):
    m = h.shape[0]

    def body(a1_ref, smf_ref, ssf_ref, smb_ref, ssb_ref, h_ref, g_ref, b_ref,
             o_ref):
        hfw = smf_ref[...] / (ssf_ref[...] + _EPS_DIV)
        hbw = smb_ref[...] / (ssb_ref[...] + _EPS_DIV)
        hn = jax.nn.relu(_ln(a1_ref[...] + hfw + hbw, g_ref[...], b_ref[...]))
        o_ref[...] = h_ref[...] + hn

    return pl.pallas_call(
        body,
        grid=(m // bm,),
        in_specs=[pl.BlockSpec((bm, 256), lambda i: (i, 0))] * 6
        + [pl.BlockSpec((1, 256), lambda i: (0, 0))] * 2,
        out_specs=pl.BlockSpec((bm, 256), lambda i: (i, 0)),
        out_shape=jax.ShapeDtypeStruct((m, 256), F32),
    )(a1h, smf, ssf, smb, ssb, h, lng.reshape(1, 256), lnb.reshape(1, 256))


def _scorer(hs, hd, e, w1a, w1b, w1c, b1, s2pad, b2, bm):
    m = hs.shape[0]

    def body(hs_ref, hd_ref, e_ref, wa_ref, wb_ref, wc_ref, b1_ref, w2_ref,
             b2_ref, o_ref):
        t = (_dot(hs_ref[...], wa_ref[...]) + _dot(hd_ref[...], wb_ref[...])
             + _dot(e_ref[...], wc_ref[...]) + b1_ref[...])
        t = jax.nn.relu(t)
        s = _dot(t, w2_ref[...]) + b2_ref[...]
        o_ref[...] = jnp.maximum(s, 0.0)

    return pl.pallas_call(
        body,
        grid=(m // bm,),
        in_specs=[pl.BlockSpec((bm, 256), lambda i: (i, 0))] * 3
        + [pl.BlockSpec((256, 256), lambda i: (0, 0))] * 3
        + [pl.BlockSpec((1, 256), lambda i: (0, 0)),
           pl.BlockSpec((256, 128), lambda i: (0, 0)),
           pl.BlockSpec((1, 128), lambda i: (0, 0))],
        out_specs=pl.BlockSpec((bm, 128), lambda i: (i, 0)),
        out_shape=jax.ShapeDtypeStruct((m, 128), F32),
    )(hs, hd, e, w1a, w1b, w1c, b1.reshape(1, 256), s2pad, b2.reshape(1, 128))


_NC = 2
_NS = 16


_GCH = 40


def _pipe(steps, issue, wait_consume):
    issue(0, 0)

    def body(j, carry):
        c0 = 2 * j
        issue(c0 + 1, 1)
        wait_consume(c0, 0)

        @pl.when(c0 + 2 < steps)
        def _():
            issue(c0 + 2, 0)

        wait_consume(c0 + 1, 1)
        return carry

    if steps // 2 > 0:
        lax.fori_loop(0, steps // 2, body, 0)
    if steps % 2:
        wait_consume(steps - 1, 0)


def _sc_gather_pair(t1, t2, i1, i2):
    nrows, d = t1.shape
    e_ = i1.shape[0]
    nw = _NC * _NS
    per_w = e_ // nw
    ch = _GCH
    steps = per_w // ch
    i1r = i1.reshape(nw, steps, ch)
    i2r = i2.reshape(nw, steps, ch)
    mesh = plsc.VectorSubcoreMesh(core_axis_name="c", subcore_axis_name="s")

    @functools.partial(
        pl.kernel,
        mesh=mesh,
        out_type=(
            jax.ShapeDtypeStruct((e_, d), F32),
            jax.ShapeDtypeStruct((e_, d), F32),
        ),
        scratch_types=[
            pltpu.VMEM((steps, ch), jnp.int32),
            pltpu.VMEM((ch, d), F32),
            pltpu.VMEM((ch, d), F32),
            pltpu.SemaphoreType.DMA,
            pltpu.SemaphoreType.DMA,
        ],
    )
    def k(t1_hbm, t2_hbm, i1_hbm, i2_hbm, o1_hbm, o2_hbm, idx_v, ra, rb,
          sema, semb):
        wid = lax.axis_index("s") * _NC + lax.axis_index("c")
        base = wid * per_w

        def gather_one(t_hbm, i_hbm, o_hbm):
            pltpu.sync_copy(i_hbm.at[wid], idx_v)
            bufs = (ra, rb)
            sems = (sema, semb)

            def issue(c, b):
                pltpu.async_copy(t_hbm.at[idx_v.at[c]], bufs[b], sems[b])

            def wait_consume(c, b):
                pltpu.make_async_copy(t_hbm.at[idx_v.at[c]], bufs[b],
                                      sems[b]).wait()
                pltpu.sync_copy(bufs[b], o_hbm.at[pl.ds(base + c * ch, ch)])

            _pipe(steps, issue, wait_consume)

        gather_one(t1_hbm, i1_hbm, o1_hbm)
        gather_one(t2_hbm, i2_hbm, o2_hbm)

    return k(t1, t2, i1r, i2r)


def _segsum3(vals_a, idx_a, vals_b, idx_b, vals_c, idx_c, n_out, zeros_half):
    d = vals_a.shape[1]
    half = d // 2
    ch = 80
    steps_a = vals_a.shape[0] // _NS // ch
    steps_b = vals_b.shape[0] // _NS // ch
    steps_c = vals_c.shape[0] // _NS // ch
    steps_mx = max(steps_a, steps_b, steps_c)
    idx_ar = idx_a.reshape(_NS, steps_a, ch)
    idx_br = idx_b.reshape(_NS, steps_b, ch)
    idx_cr = idx_c.reshape(_NS, steps_c, ch)
    row_step = 624
    rows_t = 640
    assert (_NS - 1) * row_step + rows_t == n_out
    mesh = plsc.VectorSubcoreMesh(core_axis_name="c", subcore_axis_name="s")

    @functools.partial(
        pl.kernel,
        mesh=mesh,
        out_type=jax.ShapeDtypeStruct((n_out, d), F32),
        scratch_types=[
            pltpu.VMEM((steps_mx, ch), jnp.int32),
            pltpu.VMEM((ch, half), F32),
            pltpu.VMEM((ch, half), F32),
            pltpu.VMEM_SHARED((n_out, half), F32),
            pltpu.SemaphoreType.DMA,
            pltpu.SemaphoreType.DMA,
        ],
    )
    def k(va_hbm, ia_hbm, vb_hbm, ib_hbm, vc_hbm, ic_hbm, z_hbm, out_hbm,
          idx_v, va, vb, acc, sema, semb):
        c = lax.axis_index("c")
        s = lax.axis_index("s")
        r0 = s * row_step
        col0 = c * half
        pltpu.sync_copy(z_hbm.at[pl.ds(r0, rows_t)], acc.at[pl.ds(r0, rows_t)])
        plsc.subcore_barrier()
        bufs = (va, vb)
        sems = (sema, semb)

        def accumulate(vals_hbm, i_hbm, steps):
            pltpu.sync_copy(i_hbm.at[s], idx_v.at[pl.ds(0, steps)])
            per_t = steps * ch
            base = s * per_t

            def vslice(cc):
                return vals_hbm.at[pl.ds(base + cc * ch, ch),
                                   pl.ds(col0, half)]

            def issue(cc, b):
                pltpu.async_copy(vslice(cc), bufs[b], sems[b])

            def wait_consume(cc, b):
                pltpu.make_async_copy(vslice(cc), bufs[b], sems[b]).wait()
                pltpu.sync_copy(bufs[b], acc.at[idx_v.at[cc]], add=True)

            _pipe(steps, issue, wait_consume)

        accumulate(va_hbm, ia_hbm, steps_a)
        accumulate(vb_hbm, ib_hbm, steps_b)
        accumulate(vc_hbm, ic_hbm, steps_c)
        plsc.subcore_barrier()
        pltpu.sync_copy(acc.at[pl.ds(r0, rows_t)],
                        out_hbm.at[pl.ds(r0, rows_t), pl.ds(col0, half)])

    return k(vals_a, idx_ar, vals_b, idx_br, vals_c, idx_cr, zeros_half)


def kernel(x, edge_attr, edge_index, graph_attr, params):
    del graph_attr
    p = params
    n = x.shape[0]
    bounds = [(0, 53760), (53760, 107520), (107520, 160000)]
    srcs = [edge_index[0, lo:hi] for lo, hi in bounds]
    dsts = [edge_index[1, lo:hi] for lo, hi in bounds]

    bm_n = 400
    bm_e = 320

    h = _mlp2(x, p["W11_w"].T, p["W11_b"], p["W12_w"].T, p["W12_b"], bm_n)
    es = [_mlp2(edge_attr[lo:hi], p["W21_w"].T, p["W21_b"], p["W22_w"].T,
                p["W22_b"], bm_e) for lo, hi in bounds]

    zeros_half = jnp.zeros((n, 128), F32)
    for l in range(3):
        pre = "L%d_" % l
        wcat_t = jnp.concatenate(
            [p[pre + nm + "_w"] for nm in
             ["A1", "A2", "B2", "B3", "A3", "B2", "B3"]], axis=0).T
        bcat = jnp.concatenate(
            [p[pre + nm + "_b"] for nm in
             ["A1", "A2", "B2", "B3", "A3", "B2", "B3"]], axis=0)
        a1h, t_src, t_dst = _node_cat(h, wcat_t, bcat, bm_n)
        efs, mfs, sfs, mbs, sbs = [], [], [], [], []
        for k in range(3):
            gs_k, gd_k = _sc_gather_pair(t_src, t_dst, srcs[k], dsts[k])
            ef, mf, sf, mb, sb = _edge_layer(
                es[k], gs_k, gd_k, p[pre + "B1_w"].T, p[pre + "B1_b"],
                p[pre + "lne_g"], p[pre + "lne_b"], bm_e)
            efs.append(ef)
            mfs.append(mf)
            sfs.append(sf)
            mbs.append(mb)
            sbs.append(sb)
        smf = _segsum3(mfs[0], dsts[0], mfs[1], dsts[1], mfs[2], dsts[2],
                       n, zeros_half)
        ssf = _segsum3(sfs[0], dsts[0], sfs[1], dsts[1], sfs[2], dsts[2],
                       n, zeros_half)
        smb = _segsum3(mbs[0], srcs[0], mbs[1], srcs[1], mbs[2], srcs[2],
                       n, zeros_half)
        ssb = _segsum3(sbs[0], srcs[0], sbs[1], srcs[1], sbs[2], srcs[2],
                       n, zeros_half)
        h = _node_update(a1h, smf, ssf, smb, ssb, h,
                         p[pre + "lnh_g"], p[pre + "lnh_b"], bm_n)
        es = efs

    s1t = p["s1_w"].T
    s2pad = jnp.zeros((256, 128), F32).at[:, 0].set(p["s2_w"][0])
    b2pad = jnp.zeros((128,), F32) + p["s2_b"][0]
    scores = []
    for k in range(3):
        hs_k, hd_k = _sc_gather_pair(h, h, srcs[k], dsts[k])
        scores.append(_scorer(hs_k, hd_k, es[k], s1t[0:256], s1t[256:512],
                              s1t[512:768], p["s1_b"], s2pad, b2pad, bm_e))
    return jnp.concatenate([s[:, 0:1] for s in scores], axis=0)


# speedup vs baseline: 3.0123x; 1.0691x over previous
import functools

import jax
import jax.numpy as jnp
from jax import lax
from jax.experimental import pallas as pl
from jax.experimental.pallas import tpu as pltpu
from jax.experimental.pallas import tpu_sc as plsc

F32 = jnp.float32
_EPS_LN = 1e-5
_EPS_DIV = 1e-6


def _ln(v, g, b):
    mu = jnp.mean(v, axis=-1, keepdims=True)
    var = jnp.mean((v - mu) ** 2, axis=-1, keepdims=True)
    return (v - mu) * lax.rsqrt(var + _EPS_LN) * g + b


def _dot(a, b):
    return jnp.dot(a, b, preferred_element_type=F32)


def _mlp2(x, w1t, b1, w2t, b2, bm):
    m, k = x.shape
    f1 = w1t.shape[1]
    f2 = w2t.shape[1]

    def body(x_ref, w1_ref, b1_ref, w2_ref, b2_ref, o_ref):
        t = jax.nn.relu(_dot(x_ref[...], w1_ref[...]) + b1_ref[...])
        o_ref[...] = _dot(t, w2_ref[...]) + b2_ref[...]

    return pl.pallas_call(
        body,
        grid=(m // bm,),
        in_specs=[
            pl.BlockSpec((bm, k), lambda i: (i, 0)),
            pl.BlockSpec((k, f1), lambda i: (0, 0)),
            pl.BlockSpec((1, f1), lambda i: (0, 0)),
            pl.BlockSpec((f1, f2), lambda i: (0, 0)),
            pl.BlockSpec((1, f2), lambda i: (0, 0)),
        ],
        out_specs=pl.BlockSpec((bm, f2), lambda i: (i, 0)),
        out_shape=jax.ShapeDtypeStruct((m, f2), F32),
    )(x, w1t, b1.reshape(1, f1), w2t, b2.reshape(1, f2))


def _node_cat(h, wcat_t, bcat, bm):
    m, k = h.shape

    def body(h_ref, w_ref, b_ref, a1_ref, ts_ref, td_ref):
        acc = _dot(h_ref[...], w_ref[...]) + b_ref[...]
        a1_ref[...] = acc[:, 0:256]
        ts_ref[...] = acc[:, 256:1024]
        td_ref[...] = acc[:, 1024:1792]

    return pl.pallas_call(
        body,
        grid=(m // bm,),
        in_specs=[
            pl.BlockSpec((bm, k), lambda i: (i, 0)),
            pl.BlockSpec((k, 1792), lambda i: (0, 0)),
            pl.BlockSpec((1, 1792), lambda i: (0, 0)),
        ],
        out_specs=[
            pl.BlockSpec((bm, 256), lambda i: (i, 0)),
            pl.BlockSpec((bm, 768), lambda i: (i, 0)),
            pl.BlockSpec((bm, 768), lambda i: (i, 0)),
        ],
        out_shape=[
            jax.ShapeDtypeStruct((m, 256), F32),
            jax.ShapeDtypeStruct((m, 768), F32),
            jax.ShapeDtypeStruct((m, 768), F32),
        ],
    )(h, wcat_t, bcat.reshape(1, 1792))


def _edge_layer(e, gs, gd, b1t, b1b, lng, lnb, bm):
    m = e.shape[0]

    def body(e_ref, gs_ref, gd_ref, w_ref, b_ref, g_ref, gb_ref,
             efw_ref, mfw_ref, sfw_ref, mbw_ref, sbw_ref):
        ev = e_ref[...]
        gsv = gs_ref[...]
        gdv = gd_ref[...]
        b1h = _dot(ev, w_ref[...]) + b_ref[...]
        g = g_ref[...]
        gb = gb_ref[...]
        fw = ev + jax.nn.relu(_ln(b1h + gsv[:, 256:512] + gdv[:, 512:768], g, gb))
        bw = ev + jax.nn.relu(_ln(b1h + gdv[:, 256:512] + gsv[:, 512:768], g, gb))
        sfw = jax.nn.sigmoid(fw)
        sbw = jax.nn.sigmoid(bw)
        efw_ref[...] = fw
        mfw_ref[...] = gsv[:, 0:256] * sfw
        sfw_ref[...] = sfw
        mbw_ref[...] = gdv[:, 0:256] * sbw
        sbw_ref[...] = sbw

    outs = pl.pallas_call(
        body,
        grid=(m // bm,),
        in_specs=[
            pl.BlockSpec((bm, 256), lambda i: (i, 0)),
            pl.BlockSpec((bm, 768), lambda i: (i, 0)),
            pl.BlockSpec((bm, 768), lambda i: (i, 0)),
            pl.BlockSpec((256, 256), lambda i: (0, 0)),
            pl.BlockSpec((1, 256), lambda i: (0, 0)),
            pl.BlockSpec((1, 256), lambda i: (0, 0)),
            pl.BlockSpec((1, 256), lambda i: (0, 0)),
        ],
        out_specs=[pl.BlockSpec((bm, 256), lambda i: (i, 0))] * 5,
        out_shape=[jax.ShapeDtypeStruct((m, 256), F32)] * 5,
    )(e, gs, gd, b1t, b1b.reshape(1, 256), lng.reshape(1, 256),
      lnb.reshape(1, 256))
    return outs


def _node_update(a1h, smf, ssf, smb, ssb, h, lng, lnb, bm):
    m = h.shape[0]

    def body(a1_ref, smf_ref, ssf_ref, smb_ref, ssb_ref, h_ref, g_ref, b_ref,
             o_ref):
        hfw = smf_ref[...] / (ssf_ref[...] + _EPS_DIV)
        hbw = smb_ref[...] / (ssb_ref[...] + _EPS_DIV)
        hn = jax.nn.relu(_ln(a1_ref[...] + hfw + hbw, g_ref[...], b_ref[...]))
        o_ref[...] = h_ref[...] + hn

    return pl.pallas_call(
        body,
        grid=(m // bm,),
        in_specs=[pl.BlockSpec((bm, 256), lambda i: (i, 0))] * 6
        + [pl.BlockSpec((1, 256), lambda i: (0, 0))] * 2,
        out_specs=pl.BlockSpec((bm, 256), lambda i: (i, 0)),
        out_shape=jax.ShapeDtypeStruct((m, 256), F32),
    )(a1h, smf, ssf, smb, ssb, h, lng.reshape(1, 256), lnb.reshape(1, 256))


def _scorer(hs, hd, e, w1a, w1b, w1c, b1, s2pad, b2, bm):
    m = hs.shape[0]

    def body(hs_ref, hd_ref, e_ref, wa_ref, wb_ref, wc_ref, b1_ref, w2_ref,
             b2_ref, o_ref):
        t = (_dot(hs_ref[...], wa_ref[...]) + _dot(hd_ref[...], wb_ref[...])
             + _dot(e_ref[...], wc_ref[...]) + b1_ref[...])
        t = jax.nn.relu(t)
        s = _dot(t, w2_ref[...]) + b2_ref[...]
        o_ref[...] = jnp.maximum(s, 0.0)

    return pl.pallas_call(
        body,
        grid=(m // bm,),
        in_specs=[pl.BlockSpec((bm, 256), lambda i: (i, 0))] * 3
        + [pl.BlockSpec((256, 256), lambda i: (0, 0))] * 3
        + [pl.BlockSpec((1, 256), lambda i: (0, 0)),
           pl.BlockSpec((256, 128), lambda i: (0, 0)),
           pl.BlockSpec((1, 128), lambda i: (0, 0))],
        out_specs=pl.BlockSpec((bm, 128), lambda i: (i, 0)),
        out_shape=jax.ShapeDtypeStruct((m, 128), F32),
    )(hs, hd, e, w1a, w1b, w1c, b1.reshape(1, 256), s2pad, b2.reshape(1, 128))


_NC = 2
_NS = 16


_GCH = 40


def _pipe(steps, issue, wait_consume):
    issue(0, 0)

    def body(j, carry):
        c0 = 2 * j
        issue(c0 + 1, 1)
        wait_consume(c0, 0)

        @pl.when(c0 + 2 < steps)
        def _():
            issue(c0 + 2, 0)

        wait_consume(c0 + 1, 1)
        return carry

    if steps // 2 > 0:
        lax.fori_loop(0, steps // 2, body, 0)
    if steps % 2:
        wait_consume(steps - 1, 0)


def _sc_gather_pair(t1, t2, i1, i2):
    nrows, d = t1.shape
    e_ = i1.shape[0]
    nw = _NC * _NS
    per_w = e_ // nw
    ch = _GCH
    steps = per_w // ch
    i1r = i1.reshape(nw, steps, ch)
    i2r = i2.reshape(nw, steps, ch)
    mesh = plsc.VectorSubcoreMesh(core_axis_name="c", subcore_axis_name="s")

    @functools.partial(
        pl.kernel,
        mesh=mesh,
        out_type=(
            jax.ShapeDtypeStruct((e_, d), F32),
            jax.ShapeDtypeStruct((e_, d), F32),
        ),
        scratch_types=[
            pltpu.VMEM((steps, ch), jnp.int32),
            pltpu.VMEM((ch, d), F32),
            pltpu.VMEM((ch, d), F32),
            pltpu.SemaphoreType.DMA,
            pltpu.SemaphoreType.DMA,
        ],
    )
    def k(t1_hbm, t2_hbm, i1_hbm, i2_hbm, o1_hbm, o2_hbm, idx_v, ra, rb,
          sema, semb):
        wid = lax.axis_index("s") * _NC + lax.axis_index("c")
        base = wid * per_w

        def gather_one(t_hbm, i_hbm, o_hbm):
            pltpu.sync_copy(i_hbm.at[wid], idx_v)
            bufs = (ra, rb)
            sems = (sema, semb)

            def issue(c, b):
                pltpu.async_copy(t_hbm.at[idx_v.at[c]], bufs[b], sems[b])

            def wait_consume(c, b):
                pltpu.make_async_copy(t_hbm.at[idx_v.at[c]], bufs[b],
                                      sems[b]).wait()
                pltpu.sync_copy(bufs[b], o_hbm.at[pl.ds(base + c * ch, ch)])

            _pipe(steps, issue, wait_consume)

        gather_one(t1_hbm, i1_hbm, o1_hbm)
        gather_one(t2_hbm, i2_hbm, o2_hbm)

    return k(t1, t2, i1r, i2r)


def _segsum3(vals_a, idx_a, vals_b, idx_b, vals_c, idx_c, n_out, zeros_half):
    d = vals_a.shape[1]
    half = d // 2
    ch = 80
    steps_a = vals_a.shape[0] // _NS // ch
    steps_b = vals_b.shape[0] // _NS // ch
    steps_c = vals_c.shape[0] // _NS // ch
    steps_mx = max(steps_a, steps_b, steps_c)
    idx_ar = idx_a.reshape(_NS, steps_a, ch)
    idx_br = idx_b.reshape(_NS, steps_b, ch)
    idx_cr = idx_c.reshape(_NS, steps_c, ch)
    row_step = 624
    rows_t = 640
    assert (_NS - 1) * row_step + rows_t == n_out
    mesh = plsc.VectorSubcoreMesh(core_axis_name="c", subcore_axis_name="s")

    @functools.partial(
        pl.kernel,
        mesh=mesh,
        out_type=jax.ShapeDtypeStruct((n_out, d), F32),
        scratch_types=[
            pltpu.VMEM((steps_mx, ch), jnp.int32),
            pltpu.VMEM((ch, half), F32),
            pltpu.VMEM((ch, half), F32),
            pltpu.VMEM_SHARED((n_out, half), F32),
            pltpu.SemaphoreType.DMA,
            pltpu.SemaphoreType.DMA,
        ],
    )
    def k(va_hbm, ia_hbm, vb_hbm, ib_hbm, vc_hbm, ic_hbm, z_hbm, out_hbm,
          idx_v, va, vb, acc, sema, semb):
        c = lax.axis_index("c")
        s = lax.axis_index("s")
        r0 = s * row_step
        col0 = c * half
        pltpu.sync_copy(z_hbm.at[pl.ds(r0, rows_t)], acc.at[pl.ds(r0, rows_t)])
        plsc.subcore_barrier()
        bufs = (va, vb)
        sems = (sema, semb)

        def accumulate(vals_hbm, i_hbm, steps):
            pltpu.sync_copy(i_hbm.at[s], idx_v.at[pl.ds(0, steps)])
            per_t = steps * ch
            base = s * per_t

            def vslice(cc):
                return vals_hbm.at[pl.ds(base + cc * ch, ch),
                                   pl.ds(col0, half)]

            def issue(cc, b):
                pltpu.async_copy(vslice(cc), bufs[b], sems[b])

            def wait_consume(cc, b):
                pltpu.make_async_copy(vslice(cc), bufs[b], sems[b]).wait()
                pltpu.sync_copy(bufs[b], acc.at[idx_v.at[cc]], add=True)

            _pipe(steps, issue, wait_consume)

        accumulate(va_hbm, ia_hbm, steps_a)
        accumulate(vb_hbm, ib_hbm, steps_b)
        accumulate(vc_hbm, ic_hbm, steps_c)
        plsc.subcore_barrier()
        pltpu.sync_copy(acc.at[pl.ds(r0, rows_t)],
                        out_hbm.at[pl.ds(r0, rows_t), pl.ds(col0, half)])

    return k(vals_a, idx_ar, vals_b, idx_br, vals_c, idx_cr, zeros_half)


def kernel(x, edge_attr, edge_index, graph_attr, params):
    del graph_attr
    p = params
    n = x.shape[0]
    bounds = [(0, 53760), (53760, 107520), (107520, 160000)]
    srcs = [edge_index[0, lo:hi] for lo, hi in bounds]
    dsts = [edge_index[1, lo:hi] for lo, hi in bounds]

    bm_n = 400
    bm_e = 640

    h = _mlp2(x, p["W11_w"].T, p["W11_b"], p["W12_w"].T, p["W12_b"], bm_n)
    es = [_mlp2(edge_attr[lo:hi], p["W21_w"].T, p["W21_b"], p["W22_w"].T,
                p["W22_b"], bm_e) for lo, hi in bounds]

    zeros_half = jnp.zeros((n, 128), F32)
    for l in range(3):
        pre = "L%d_" % l
        wcat_t = jnp.concatenate(
            [p[pre + nm + "_w"] for nm in
             ["A1", "A2", "B2", "B3", "A3", "B2", "B3"]], axis=0).T
        bcat = jnp.concatenate(
            [p[pre + nm + "_b"] for nm in
             ["A1", "A2", "B2", "B3", "A3", "B2", "B3"]], axis=0)
        a1h, t_src, t_dst = _node_cat(h, wcat_t, bcat, bm_n)
        efs, mfs, sfs, mbs, sbs = [], [], [], [], []
        for k in range(3):
            gs_k, gd_k = _sc_gather_pair(t_src, t_dst, srcs[k], dsts[k])
            ef, mf, sf, mb, sb = _edge_layer(
                es[k], gs_k, gd_k, p[pre + "B1_w"].T, p[pre + "B1_b"],
                p[pre + "lne_g"], p[pre + "lne_b"], bm_e)
            efs.append(ef)
            mfs.append(mf)
            sfs.append(sf)
            mbs.append(mb)
            sbs.append(sb)
        smf = _segsum3(mfs[0], dsts[0], mfs[1], dsts[1], mfs[2], dsts[2],
                       n, zeros_half)
        ssf = _segsum3(sfs[0], dsts[0], sfs[1], dsts[1], sfs[2], dsts[2],
                       n, zeros_half)
        smb = _segsum3(mbs[0], srcs[0], mbs[1], srcs[1], mbs[2], srcs[2],
                       n, zeros_half)
        ssb = _segsum3(sbs[0], srcs[0], sbs[1], srcs[1], sbs[2], srcs[2],
                       n, zeros_half)
        h = _node_update(a1h, smf, ssf, smb, ssb, h,
                         p[pre + "lnh_g"], p[pre + "lnh_b"], bm_n)
        es = efs

    s1t = p["s1_w"].T
    s2pad = jnp.zeros((256, 128), F32).at[:, 0].set(p["s2_w"][0])
    b2pad = jnp.zeros((128,), F32) + p["s2_b"][0]
    scores = []
    for k in range(3):
        hs_k, hd_k = _sc_gather_pair(h, h, srcs[k], dsts[k])
        scores.append(_scorer(hs_k, hd_k, es[k], s1t[0:256], s1t[256:512],
                              s1t[512:768], p["s1_b"], s2pad, b2pad, bm_e))
    return jnp.concatenate([s[:, 0:1] for s in scores], axis=0)


# speedup vs baseline: 3.0968x; 1.0281x over previous
import functools

import jax
import jax.numpy as jnp
from jax import lax
from jax.experimental import pallas as pl
from jax.experimental.pallas import tpu as pltpu
from jax.experimental.pallas import tpu_sc as plsc

F32 = jnp.float32
_EPS_LN = 1e-5
_EPS_DIV = 1e-6


def _ln(v, g, b):
    mu = jnp.mean(v, axis=-1, keepdims=True)
    var = jnp.mean((v - mu) ** 2, axis=-1, keepdims=True)
    return (v - mu) * lax.rsqrt(var + _EPS_LN) * g + b


def _dot(a, b):
    return jnp.dot(a, b, preferred_element_type=F32)


def _mlp2(x, w1t, b1, w2t, b2, bm):
    m, k = x.shape
    f1 = w1t.shape[1]
    f2 = w2t.shape[1]

    def body(x_ref, w1_ref, b1_ref, w2_ref, b2_ref, o_ref):
        t = jax.nn.relu(_dot(x_ref[...], w1_ref[...]) + b1_ref[...])
        o_ref[...] = _dot(t, w2_ref[...]) + b2_ref[...]

    return pl.pallas_call(
        body,
        grid=(m // bm,),
        in_specs=[
            pl.BlockSpec((bm, k), lambda i: (i, 0)),
            pl.BlockSpec((k, f1), lambda i: (0, 0)),
            pl.BlockSpec((1, f1), lambda i: (0, 0)),
            pl.BlockSpec((f1, f2), lambda i: (0, 0)),
            pl.BlockSpec((1, f2), lambda i: (0, 0)),
        ],
        out_specs=pl.BlockSpec((bm, f2), lambda i: (i, 0)),
        out_shape=jax.ShapeDtypeStruct((m, f2), F32),
    )(x, w1t, b1.reshape(1, f1), w2t, b2.reshape(1, f2))


def _node_cat(h, wcat_t, bcat, bm):
    m, k = h.shape

    def body(h_ref, w_ref, b_ref, a1_ref, ts_ref, td_ref):
        acc = _dot(h_ref[...], w_ref[...]) + b_ref[...]
        a1_ref[...] = acc[:, 0:256]
        ts_ref[...] = acc[:, 256:1024]
        td_ref[...] = acc[:, 1024:1792]

    return pl.pallas_call(
        body,
        grid=(m // bm,),
        in_specs=[
            pl.BlockSpec((bm, k), lambda i: (i, 0)),
            pl.BlockSpec((k, 1792), lambda i: (0, 0)),
            pl.BlockSpec((1, 1792), lambda i: (0, 0)),
        ],
        out_specs=[
            pl.BlockSpec((bm, 256), lambda i: (i, 0)),
            pl.BlockSpec((bm, 768), lambda i: (i, 0)),
            pl.BlockSpec((bm, 768), lambda i: (i, 0)),
        ],
        out_shape=[
            jax.ShapeDtypeStruct((m, 256), F32),
            jax.ShapeDtypeStruct((m, 768), F32),
            jax.ShapeDtypeStruct((m, 768), F32),
        ],
    )(h, wcat_t, bcat.reshape(1, 1792))


def _edge_layer(e, gs, gd, b1t, b1b, lng, lnb, bm):
    m = e.shape[0]

    def body(e_ref, gs_ref, gd_ref, w_ref, b_ref, g_ref, gb_ref,
             efw_ref, mfw_ref, sfw_ref, mbw_ref, sbw_ref):
        ev = e_ref[...]
        gsv = gs_ref[...]
        gdv = gd_ref[...]
        b1h = _dot(ev, w_ref[...]) + b_ref[...]
        g = g_ref[...]
        gb = gb_ref[...]
        fw = ev + jax.nn.relu(_ln(b1h + gsv[:, 256:512] + gdv[:, 512:768], g, gb))
        bw = ev + jax.nn.relu(_ln(b1h + gdv[:, 256:512] + gsv[:, 512:768], g, gb))
        sfw = jax.nn.sigmoid(fw)
        sbw = jax.nn.sigmoid(bw)
        efw_ref[...] = fw
        mfw_ref[...] = gsv[:, 0:256] * sfw
        sfw_ref[...] = sfw
        mbw_ref[...] = gdv[:, 0:256] * sbw
        sbw_ref[...] = sbw

    outs = pl.pallas_call(
        body,
        grid=(m // bm,),
        in_specs=[
            pl.BlockSpec((bm, 256), lambda i: (i, 0)),
            pl.BlockSpec((bm, 768), lambda i: (i, 0)),
            pl.BlockSpec((bm, 768), lambda i: (i, 0)),
            pl.BlockSpec((256, 256), lambda i: (0, 0)),
            pl.BlockSpec((1, 256), lambda i: (0, 0)),
            pl.BlockSpec((1, 256), lambda i: (0, 0)),
            pl.BlockSpec((1, 256), lambda i: (0, 0)),
        ],
        out_specs=[pl.BlockSpec((bm, 256), lambda i: (i, 0))] * 5,
        out_shape=[jax.ShapeDtypeStruct((m, 256), F32)] * 5,
    )(e, gs, gd, b1t, b1b.reshape(1, 256), lng.reshape(1, 256),
      lnb.reshape(1, 256))
    return outs


def _node_update(a1h, smf, ssf, smb, ssb, h, lng, lnb, bm):
    m = h.shape[0]

    def body(a1_ref, smf_ref, ssf_ref, smb_ref, ssb_ref, h_ref, g_ref, b_ref,
             o_ref):
        hfw = smf_ref[...] / (ssf_ref[...] + _EPS_DIV)
        hbw = smb_ref[...] / (ssb_ref[...] + _EPS_DIV)
        hn = jax.nn.relu(_ln(a1_ref[...] + hfw + hbw, g_ref[...], b_ref[...]))
        o_ref[...] = h_ref[...] + hn

    return pl.pallas_call(
        body,
        grid=(m // bm,),
        in_specs=[pl.BlockSpec((bm, 256), lambda i: (i, 0))] * 6
        + [pl.BlockSpec((1, 256), lambda i: (0, 0))] * 2,
        out_specs=pl.BlockSpec((bm, 256), lambda i: (i, 0)),
        out_shape=jax.ShapeDtypeStruct((m, 256), F32),
    )(a1h, smf, ssf, smb, ssb, h, lng.reshape(1, 256), lnb.reshape(1, 256))


def _scorer(hs, hd, e, w1a, w1b, w1c, b1, s2pad, b2, bm):
    m = hs.shape[0]

    def body(hs_ref, hd_ref, e_ref, wa_ref, wb_ref, wc_ref, b1_ref, w2_ref,
             b2_ref, o_ref):
        t = (_dot(hs_ref[...], wa_ref[...]) + _dot(hd_ref[...], wb_ref[...])
             + _dot(e_ref[...], wc_ref[...]) + b1_ref[...])
        t = jax.nn.relu(t)
        s = _dot(t, w2_ref[...]) + b2_ref[...]
        o_ref[...] = jnp.maximum(s, 0.0)

    return pl.pallas_call(
        body,
        grid=(m // bm,),
        in_specs=[pl.BlockSpec((bm, 256), lambda i: (i, 0))] * 3
        + [pl.BlockSpec((256, 256), lambda i: (0, 0))] * 3
        + [pl.BlockSpec((1, 256), lambda i: (0, 0)),
           pl.BlockSpec((256, 128), lambda i: (0, 0)),
           pl.BlockSpec((1, 128), lambda i: (0, 0))],
        out_specs=pl.BlockSpec((bm, 128), lambda i: (i, 0)),
        out_shape=jax.ShapeDtypeStruct((m, 128), F32),
    )(hs, hd, e, w1a, w1b, w1c, b1.reshape(1, 256), s2pad, b2.reshape(1, 128))


_NC = 2
_NS = 16


_GCH = 40


def _pipe(steps, issue, wait_consume):
    issue(0, 0)

    def body(j, carry):
        c0 = 2 * j
        issue(c0 + 1, 1)
        wait_consume(c0, 0)

        @pl.when(c0 + 2 < steps)
        def _():
            issue(c0 + 2, 0)

        wait_consume(c0 + 1, 1)
        return carry

    if steps // 2 > 0:
        lax.fori_loop(0, steps // 2, body, 0)
    if steps % 2:
        wait_consume(steps - 1, 0)


def _sc_gather_pair(t1, t2, i1, i2):
    nrows, d = t1.shape
    e_ = i1.shape[0]
    nw = _NC * _NS
    per_w = e_ // nw
    ch = _GCH
    steps = per_w // ch
    i1r = i1.reshape(nw, steps, ch)
    i2r = i2.reshape(nw, steps, ch)
    mesh = plsc.VectorSubcoreMesh(core_axis_name="c", subcore_axis_name="s")

    @functools.partial(
        pl.kernel,
        mesh=mesh,
        out_type=(
            jax.ShapeDtypeStruct((e_, d), F32),
            jax.ShapeDtypeStruct((e_, d), F32),
        ),
        scratch_types=[
            pltpu.VMEM((steps, ch), jnp.int32),
            pltpu.VMEM((ch, d), F32),
            pltpu.VMEM((ch, d), F32),
            pltpu.SemaphoreType.DMA,
            pltpu.SemaphoreType.DMA,
        ],
    )
    def k(t1_hbm, t2_hbm, i1_hbm, i2_hbm, o1_hbm, o2_hbm, idx_v, ra, rb,
          sema, semb):
        wid = lax.axis_index("s") * _NC + lax.axis_index("c")
        base = wid * per_w

        def gather_one(t_hbm, i_hbm, o_hbm):
            pltpu.sync_copy(i_hbm.at[wid], idx_v)
            bufs = (ra, rb)
            sems = (sema, semb)

            def issue(c, b):
                pltpu.async_copy(t_hbm.at[idx_v.at[c]], bufs[b], sems[b])

            def wait_consume(c, b):
                pltpu.make_async_copy(t_hbm.at[idx_v.at[c]], bufs[b],
                                      sems[b]).wait()
                pltpu.sync_copy(bufs[b], o_hbm.at[pl.ds(base + c * ch, ch)])

            _pipe(steps, issue, wait_consume)

        gather_one(t1_hbm, i1_hbm, o1_hbm)
        gather_one(t2_hbm, i2_hbm, o2_hbm)

    return k(t1, t2, i1r, i2r)


def _segsum3(vals_a, idx_a, vals_b, idx_b, vals_c, idx_c, n_out, zeros_half):
    d = vals_a.shape[1]
    half = d // 2
    ch = 80
    steps_a = vals_a.shape[0] // _NS // ch
    steps_b = vals_b.shape[0] // _NS // ch
    steps_c = vals_c.shape[0] // _NS // ch
    steps_mx = max(steps_a, steps_b, steps_c)
    idx_ar = idx_a.reshape(_NS, steps_a, ch)
    idx_br = idx_b.reshape(_NS, steps_b, ch)
    idx_cr = idx_c.reshape(_NS, steps_c, ch)
    row_step = 624
    rows_t = 640
    assert (_NS - 1) * row_step + rows_t == n_out
    mesh = plsc.VectorSubcoreMesh(core_axis_name="c", subcore_axis_name="s")

    @functools.partial(
        pl.kernel,
        mesh=mesh,
        out_type=jax.ShapeDtypeStruct((n_out, d), F32),
        scratch_types=[
            pltpu.VMEM((steps_mx, ch), jnp.int32),
            pltpu.VMEM((ch, half), F32),
            pltpu.VMEM((ch, half), F32),
            pltpu.VMEM_SHARED((n_out, half), F32),
            pltpu.SemaphoreType.DMA,
            pltpu.SemaphoreType.DMA,
        ],
    )
    def k(va_hbm, ia_hbm, vb_hbm, ib_hbm, vc_hbm, ic_hbm, z_hbm, out_hbm,
          idx_v, va, vb, acc, sema, semb):
        c = lax.axis_index("c")
        s = lax.axis_index("s")
        r0 = s * row_step
        col0 = c * half
        pltpu.sync_copy(z_hbm.at[pl.ds(r0, rows_t)], acc.at[pl.ds(r0, rows_t)])
        plsc.subcore_barrier()
        bufs = (va, vb)
        sems = (sema, semb)

        def accumulate(vals_hbm, i_hbm, steps):
            pltpu.sync_copy(i_hbm.at[s], idx_v.at[pl.ds(0, steps)])
            per_t = steps * ch
            base = s * per_t

            def vslice(cc):
                return vals_hbm.at[pl.ds(base + cc * ch, ch),
                                   pl.ds(col0, half)]

            def issue(cc, b):
                pltpu.async_copy(vslice(cc), bufs[b], sems[b])

            def wait_consume(cc, b):
                pltpu.make_async_copy(vslice(cc), bufs[b], sems[b]).wait()
                pltpu.sync_copy(bufs[b], acc.at[idx_v.at[cc]], add=True)

            _pipe(steps, issue, wait_consume)

        accumulate(va_hbm, ia_hbm, steps_a)
        accumulate(vb_hbm, ib_hbm, steps_b)
        accumulate(vc_hbm, ic_hbm, steps_c)
        plsc.subcore_barrier()
        pltpu.sync_copy(acc.at[pl.ds(r0, rows_t)],
                        out_hbm.at[pl.ds(r0, rows_t), pl.ds(col0, half)])

    return k(vals_a, idx_ar, vals_b, idx_br, vals_c, idx_cr, zeros_half)


def kernel(x, edge_attr, edge_index, graph_attr, params):
    del graph_attr
    p = params
    n = x.shape[0]
    bounds = [(0, 53760), (53760, 107520), (107520, 160000)]
    srcs = [edge_index[0, lo:hi] for lo, hi in bounds]
    dsts = [edge_index[1, lo:hi] for lo, hi in bounds]

    bm_n = 1000
    bm_e = 1280

    h = _mlp2(x, p["W11_w"].T, p["W11_b"], p["W12_w"].T, p["W12_b"], bm_n)
    es = [_mlp2(edge_attr[lo:hi], p["W21_w"].T, p["W21_b"], p["W22_w"].T,
                p["W22_b"], bm_e) for lo, hi in bounds]

    zeros_half = jnp.zeros((n, 128), F32)
    for l in range(3):
        pre = "L%d_" % l
        wcat_t = jnp.concatenate(
            [p[pre + nm + "_w"] for nm in
             ["A1", "A2", "B2", "B3", "A3", "B2", "B3"]], axis=0).T
        bcat = jnp.concatenate(
            [p[pre + nm + "_b"] for nm in
             ["A1", "A2", "B2", "B3", "A3", "B2", "B3"]], axis=0)
        a1h, t_src, t_dst = _node_cat(h, wcat_t, bcat, bm_n)
        efs, mfs, sfs, mbs, sbs = [], [], [], [], []
        for k in range(3):
            gs_k, gd_k = _sc_gather_pair(t_src, t_dst, srcs[k], dsts[k])
            ef, mf, sf, mb, sb = _edge_layer(
                es[k], gs_k, gd_k, p[pre + "B1_w"].T, p[pre + "B1_b"],
                p[pre + "lne_g"], p[pre + "lne_b"], bm_e)
            efs.append(ef)
            mfs.append(mf)
            sfs.append(sf)
            mbs.append(mb)
            sbs.append(sb)
        smf = _segsum3(mfs[0], dsts[0], mfs[1], dsts[1], mfs[2], dsts[2],
                       n, zeros_half)
        ssf = _segsum3(sfs[0], dsts[0], sfs[1], dsts[1], sfs[2], dsts[2],
                       n, zeros_half)
        smb = _segsum3(mbs[0], srcs[0], mbs[1], srcs[1], mbs[2], srcs[2],
                       n, zeros_half)
        ssb = _segsum3(sbs[0], srcs[0], sbs[1], srcs[1], sbs[2], srcs[2],
                       n, zeros_half)
        h = _node_update(a1h, smf, ssf, smb, ssb, h,
                         p[pre + "lnh_g"], p[pre + "lnh_b"], bm_n)
        es = efs

    s1t = p["s1_w"].T
    s2pad = jnp.zeros((256, 128), F32).at[:, 0].set(p["s2_w"][0])
    b2pad = jnp.zeros((128,), F32) + p["s2_b"][0]
    scores = []
    for k in range(3):
        hs_k, hd_k = _sc_gather_pair(h, h, srcs[k], dsts[k])
        scores.append(_scorer(hs_k, hd_k, es[k], s1t[0:256], s1t[256:512],
                              s1t[512:768], p["s1_b"], s2pad, b2pad, bm_e))
    return jnp.concatenate([s[:, 0:1] for s in scores], axis=0)
